# Initial kernel scaffold; baseline (speedup 1.0000x reference)
import jax
import jax.numpy as jnp
from jax.experimental import pallas as pl


def kernel(inputs, training):
    raise NotImplementedError("write your pallas kernel here")


# speedup vs baseline: 7.9164x; 7.9164x over previous
import functools

import jax
import jax.numpy as jnp
import numpy as np
from jax import lax
from jax.experimental import pallas as pl
from jax.experimental.pallas import tpu as pltpu
from jax.experimental.pallas import tpu_sc as plsc

_NC = 2
_NS = 16
_L = 16
_NW = _NC * _NS

_INT_MIN = np.int32(-(2 ** 31))


def _skey_from_bits(b):
  return jnp.where(b >= 0, b, jnp.bitwise_xor(jnp.bitwise_not(b), _INT_MIN))


@functools.cache
def _make_sc_kernel(B, N):
  assert B % _NW == 0 and N % _L == 0
  rpw = B // _NW
  nv = N // _L

  mesh = plsc.VectorSubcoreMesh(
      core_axis_name="c", subcore_axis_name="s",
      num_cores=_NC, num_subcores=_NS)

  @functools.partial(
      pl.kernel,
      out_type=jax.ShapeDtypeStruct((B, N), jnp.float32),
      mesh=mesh,
      scratch_types=[
          pltpu.VMEM((N,), jnp.float32),
          pltpu.VMEM((N,), jnp.int32),
          pltpu.VMEM((_L,), jnp.int32),
          pltpu.VMEM((_L,), jnp.int32),
      ],
      compiler_params=pltpu.CompilerParams(needs_layout_passes=False),
  )
  def body(x_hbm, k_hbm, t_hbm, out_hbm, row_v, key_v, k_v, t_v):
    wid = lax.axis_index("s") * _NC + lax.axis_index("c")
    pltpu.sync_copy(t_hbm, t_v)
    tmask = t_v[...] != 0

    for r in range(rpw):
      row = wid * rpw + r
      pltpu.sync_copy(x_hbm.at[row], row_v)
      pltpu.sync_copy(k_hbm.at[row], k_v)
      kk = k_v[...]

      def pass_a(i, accs):
        a0, a1 = accs
        v0 = row_v[pl.ds((2 * i) * _L, _L)]
        v1 = row_v[pl.ds((2 * i + 1) * _L, _L)]
        key_v[pl.ds((2 * i) * _L, _L)] = _skey_from_bits(
            plsc.bitcast(v0, jnp.int32))
        key_v[pl.ds((2 * i + 1) * _L, _L)] = _skey_from_bits(
            plsc.bitcast(v1, jnp.int32))
        return a0 + v0, a1 + v1
      zf = jnp.zeros((_L,), jnp.float32)
      na, nb = plsc.parallel_loop(0, nv // 2, carry=(zf, zf))(pass_a)
      num = jnp.broadcast_to(jnp.sum(na + nb), (_L,))

      s_cand = jnp.full((_L,), _INT_MIN, jnp.int32)
      for bit in range(31, -1, -1):
        if bit == 31:
          trial = jnp.zeros((_L,), jnp.int32)
        else:
          trial = s_cand + np.int32(1 << bit)

        def pass_c(i, accs, trial=trial):
          a0, a1, a2, a3 = accs
          k0 = key_v[pl.ds((4 * i) * _L, _L)]
          k1 = key_v[pl.ds((4 * i + 1) * _L, _L)]
          k2 = key_v[pl.ds((4 * i + 2) * _L, _L)]
          k3 = key_v[pl.ds((4 * i + 3) * _L, _L)]
          a0 = a0 + plsc.all_reduce_population_count(k0 < trial)
          a1 = a1 + plsc.all_reduce_population_count(k1 < trial)
          a2 = a2 + plsc.all_reduce_population_count(k2 < trial)
          a3 = a3 + plsc.all_reduce_population_count(k3 < trial)
          return a0, a1, a2, a3

        zi = jnp.zeros((_L,), jnp.int32)
        accs = plsc.parallel_loop(
            0, nv // 4, unroll=2, carry=(zi, zi, zi, zi))(pass_c)
        cnt = accs[0] + accs[1] + accs[2] + accs[3]
        s_cand = jnp.where(cnt <= kk, trial, s_cand)

      thresh = s_cand

      def pass_b(i, accs):
        a0, a1 = accs
        v0 = row_v[pl.ds((2 * i) * _L, _L)]
        v1 = row_v[pl.ds((2 * i + 1) * _L, _L)]
        k0 = key_v[pl.ds((2 * i) * _L, _L)]
        k1 = key_v[pl.ds((2 * i + 1) * _L, _L)]
        a0 = a0 + jnp.where(k0 >= thresh, v0, np.float32(0.0))
        a1 = a1 + jnp.where(k1 >= thresh, v1, np.float32(0.0))
        return a0, a1
      da, db = plsc.parallel_loop(0, nv // 2, carry=(zf, zf))(pass_b)
      den = jnp.broadcast_to(jnp.sum(da + db), (_L,))

      scale = jnp.abs(jnp.where(den == 0.0, np.float32(0.0), num / den))

      def pass_d(i):
        v = row_v[pl.ds(i * _L, _L)]
        kv = key_v[pl.ds(i * _L, _L)]
        masked = jnp.where(kv >= thresh, v, np.float32(0.0))
        row_v[pl.ds(i * _L, _L)] = jnp.where(tmask, scale * masked, v)
      plsc.parallel_loop(0, nv)(pass_d)

      pltpu.sync_copy(row_v, out_hbm.at[row])

  return body


def kernel(inputs, training):
  B, N = inputs.shape
  probs = jax.random.uniform(
      jax.random.fold_in(jax.random.key(0), 1), (B,),
      minval=0.0, maxval=1.0)
  k = jnp.maximum(
      jnp.ceil(np.float32(N) * probs).astype(jnp.int32) - 1, 0)
  k16 = jnp.broadcast_to(k[:, None], (B, _L)).astype(jnp.int32)
  t16 = jnp.full((_L,), training, dtype=jnp.int32)
  return _make_sc_kernel(B, N)(inputs, k16, t16)


# speedup vs baseline: 15.9006x; 2.0086x over previous
import functools

import jax
import jax.numpy as jnp
import numpy as np
from jax import lax
from jax.experimental import pallas as pl
from jax.experimental.pallas import tpu as pltpu
from jax.experimental.pallas import tpu_sc as plsc

_NC = 2
_NS = 16
_L = 16
_NW = _NC * _NS

_INT_MIN = np.int32(-(2 ** 31))
_NBINS = 2048


@functools.cache
def _make_sc_kernel(B, N):
  assert B % _NW == 0 and N % _L == 0
  rpw = B // _NW
  nv = N // _L

  mesh = plsc.VectorSubcoreMesh(
      core_axis_name="c", subcore_axis_name="s",
      num_cores=_NC, num_subcores=_NS)

  @functools.partial(
      pl.kernel,
      out_type=jax.ShapeDtypeStruct((B, N), jnp.float32),
      mesh=mesh,
      scratch_types=[
          pltpu.VMEM((N,), jnp.float32),
          pltpu.VMEM((N,), jnp.int32),
          pltpu.VMEM((_NBINS,), jnp.int32),
          pltpu.VMEM((_L,), jnp.int32),
          pltpu.VMEM((_L,), jnp.int32),
      ],
      compiler_params=pltpu.CompilerParams(needs_layout_passes=False),
  )
  def body(x_hbm, k_hbm, t_hbm, out_hbm, row_v, key_v, hist_v, k_v, t_v):
    wid = lax.axis_index("s") * _NC + lax.axis_index("c")
    pltpu.sync_copy(t_hbm, t_v)
    tmask = t_v[...] != 0
    zi = jnp.zeros((_L,), jnp.int32)
    zf = jnp.zeros((_L,), jnp.float32)

    def zero_hist(i):
      hist_v[pl.ds(i * _L, _L)] = zi
    plsc.parallel_loop(0, _NBINS // _L)(zero_hist)

    def locate(nbins, kp):
      def lbody(i, carry):
        run, bacc, cbacc = carry
        h = hist_v[pl.ds(i * _L, _L)]
        hist_v[pl.ds(i * _L, _L)] = zi
        s = run + plsc.cumsum(h)
        m = s <= kp
        bacc = bacc + plsc.all_reduce_population_count(m)
        cbacc = cbacc + jnp.where(m, h, np.int32(0))
        run = run + jnp.broadcast_to(jnp.sum(h), (_L,))
        return run, bacc, cbacc
      _, b, cbacc = plsc.parallel_loop(
          0, nbins // _L, carry=(zi, zi, zi))(lbody)
      cbelow = jnp.broadcast_to(jnp.sum(cbacc), (_L,))
      return b, kp - cbelow

    for r in range(rpw):
      row = wid * rpw + r
      pltpu.sync_copy(x_hbm.at[row], row_v)
      pltpu.sync_copy(k_hbm.at[row], k_v)
      kp = k_v[...]

      def pass_a(i, acc):
        v = row_v[pl.ds(i * _L, _L)]
        b = plsc.bitcast(v, jnp.int32)
        ub = jnp.where(b >= 0, jnp.bitwise_xor(b, _INT_MIN),
                       jnp.bitwise_not(b))
        key_v[pl.ds(i * _L, _L)] = ub
        d = lax.shift_right_logical(ub, np.int32(21))
        cnts, last = plsc.scan_count(d)
        plsc.addupdate_scatter(hist_v, [d], cnts, mask=last)
        return acc + v
      na = plsc.parallel_loop(0, nv, unroll=2, carry=zf)(pass_a)
      num = jnp.broadcast_to(jnp.sum(na), (_L,))

      b1, k2 = locate(2048, kp)

      def pass_h2(i):
        ub = key_v[pl.ds(i * _L, _L)]
        m = lax.shift_right_logical(ub, np.int32(21)) == b1
        d = jnp.bitwise_and(lax.shift_right_logical(ub, np.int32(10)),
                            np.int32(0x7FF))
        cnts, last = plsc.scan_count(d, mask=m)
        plsc.addupdate_scatter(hist_v, [d], cnts, mask=last)
      plsc.parallel_loop(0, nv, unroll=2)(pass_h2)

      b2, k3 = locate(2048, k2)
      hi21 = jnp.bitwise_or(lax.shift_left(b1, np.int32(11)), b2)

      def pass_h3(i):
        ub = key_v[pl.ds(i * _L, _L)]
        m = lax.shift_right_logical(ub, np.int32(10)) == hi21
        d = jnp.bitwise_and(ub, np.int32(0x3FF))
        cnts, last = plsc.scan_count(d, mask=m)
        plsc.addupdate_scatter(hist_v, [d], cnts, mask=last)
      plsc.parallel_loop(0, nv, unroll=2)(pass_h3)

      b3, _ = locate(1024, k3)

      thresh = jnp.bitwise_xor(
          jnp.bitwise_or(lax.shift_left(hi21, np.int32(10)), b3), _INT_MIN)

      def pass_b(i, acc):
        v = row_v[pl.ds(i * _L, _L)]
        sk = jnp.bitwise_xor(key_v[pl.ds(i * _L, _L)], _INT_MIN)
        return acc + jnp.where(sk >= thresh, v, np.float32(0.0))
      da = plsc.parallel_loop(0, nv, unroll=2, carry=zf)(pass_b)
      den = jnp.broadcast_to(jnp.sum(da), (_L,))

      scale = jnp.abs(jnp.where(den == 0.0, np.float32(0.0), num / den))

      def pass_d(i):
        v = row_v[pl.ds(i * _L, _L)]
        sk = jnp.bitwise_xor(key_v[pl.ds(i * _L, _L)], _INT_MIN)
        masked = jnp.where(sk >= thresh, v, np.float32(0.0))
        row_v[pl.ds(i * _L, _L)] = jnp.where(tmask, scale * masked, v)
      plsc.parallel_loop(0, nv, unroll=2)(pass_d)

      pltpu.sync_copy(row_v, out_hbm.at[row])

  return body


def kernel(inputs, training):
  B, N = inputs.shape
  probs = jax.random.uniform(
      jax.random.fold_in(jax.random.key(0), 1), (B,),
      minval=0.0, maxval=1.0)
  k = jnp.maximum(
      jnp.ceil(np.float32(N) * probs).astype(jnp.int32) - 1, 0)
  k16 = jnp.broadcast_to(k[:, None], (B, _L)).astype(jnp.int32)
  t16 = jnp.full((_L,), training, dtype=jnp.int32)
  return _make_sc_kernel(B, N)(inputs, k16, t16)
